# Initial kernel scaffold; baseline (speedup 1.0000x reference)
import jax
import jax.numpy as jnp
from jax.experimental import pallas as pl


def kernel(x, edge_index, W1, b1, W2, b2, W3, b3):
    raise NotImplementedError("write your pallas kernel here")


# speedup vs baseline: 4.7192x; 4.7192x over previous
import functools

import jax
import jax.numpy as jnp
from jax import lax
from jax.experimental import pallas as pl
from jax.experimental.pallas import tpu as pltpu
from jax.experimental.pallas import tpu_sc as plsc

N = 10000
E = 320000
NP = 10240
K = 128
NW = 32
WPT = 80
EP = NW * WPT * K
RPT = NP // 16
ZR = 64
FC = 128
RB = 400

_mesh = plsc.VectorSubcoreMesh(core_axis_name="c", subcore_axis_name="s")


def _sc_segsum(vals, gidx_hbm, sidx_hbm, zeros_hbm):
  F = vals.shape[1]

  @functools.partial(
      pl.kernel,
      out_type=jax.ShapeDtypeStruct((2 * NP, F), jnp.float32),
      mesh=_mesh,
      scratch_types=[
          pltpu.VMEM((K,), jnp.int32),
          pltpu.VMEM((K,), jnp.int32),
          pltpu.VMEM((K, F), jnp.float32),
          pltpu.VMEM((ZR, F), jnp.float32),
          pltpu.VMEM_SHARED((NP, F), jnp.float32),
          pltpu.SemaphoreType.DMA,
      ],
  )
  def k(vals_hbm, g_hbm, s_hbm, z_hbm, out_hbm, gidx, sidx, rows, zbuf, acc,
        sem):
    cid = lax.axis_index("c")
    sid = lax.axis_index("s")
    wid = cid * 16 + sid
    pltpu.sync_copy(z_hbm, zbuf)

    @pl.loop(0, RPT // ZR)
    def _(i):
      pltpu.sync_copy(zbuf, acc.at[pl.ds(sid * RPT + i * ZR, ZR)])

    plsc.subcore_barrier()

    @pl.loop(0, WPT)
    def _(w):
      base = (wid * WPT + w) * K
      pltpu.sync_copy(g_hbm.at[pl.ds(base, K)], gidx)
      pltpu.sync_copy(s_hbm.at[pl.ds(base, K)], sidx)
      pltpu.async_copy(vals_hbm.at[gidx], rows, sem).wait()
      pltpu.sync_copy(rows, acc.at[sidx], add=True)

    plsc.subcore_barrier()
    pltpu.sync_copy(
        acc.at[pl.ds(sid * RPT, RPT)],
        out_hbm.at[pl.ds(cid * NP + sid * RPT, RPT)],
    )

  return k(vals, gidx_hbm, sidx_hbm, zeros_hbm).reshape(2, NP, F)


def _tc_prep(x, dp):
  def body(x_ref, dp_ref, dis_ref, hs_ref):
    p = dp_ref[...]
    s = p[0][:, 0:16] + p[1][:, 0:16]
    d16 = jnp.where(s > 0.0, lax.rsqrt(s), 0.0)
    dis_ref[...] = d16
    hs_ref[...] = x_ref[...] * d16[:, 0:1]

  return pl.pallas_call(
      body,
      grid=(N // RB,),
      in_specs=[
          pl.BlockSpec((RB, 128), lambda i: (i, 0)),
          pl.BlockSpec((2, RB, 128), lambda i: (0, i, 0)),
      ],
      out_specs=[
          pl.BlockSpec((RB, 16), lambda i: (i, 0)),
          pl.BlockSpec((RB, 128), lambda i: (i, 0)),
      ],
      out_shape=[
          jax.ShapeDtypeStruct((N, 16), jnp.float32),
          jax.ShapeDtypeStruct((N, 128), jnp.float32),
      ],
  )(x, dp)


def _tc_mid(partials, dis16):
  C = len(partials)

  def body(*refs):
    p_refs = refs[:C]
    dis_ref = refs[C]
    du_ref, v2_ref = refs[C + 1:]
    d = dis_ref[...][:, 0:1]
    dus = []
    for pr in p_refs:
      p = pr[...]
      dus.append(d * (p[0] + p[1]))
    du_ref[...] = jnp.concatenate(dus, axis=1)
    v2_ref[...] = jnp.stack([d * du for du in dus], axis=0)

  return pl.pallas_call(
      body,
      grid=(N // RB,),
      in_specs=[pl.BlockSpec((2, RB, FC), lambda i: (0, i, 0))] * C
      + [pl.BlockSpec((RB, 16), lambda i: (i, 0))],
      out_specs=[
          pl.BlockSpec((RB, C * FC), lambda i: (i, 0)),
          pl.BlockSpec((C, RB, FC), lambda i: (0, i, 0)),
      ],
      out_shape=[
          jax.ShapeDtypeStruct((N, C * FC), jnp.float32),
          jax.ShapeDtypeStruct((C, N, FC), jnp.float32),
      ],
  )(*partials, dis16)


def _tc_layer_out(h, du1, partials2, dis16, Wl, bl, emit_hs):
  C = len(partials2)
  Fin = h.shape[1]
  Fout = Wl.shape[2]
  C2 = Fout // FC

  def body(*refs):
    h_ref, du_ref = refs[0], refs[1]
    p_refs = refs[2:2 + C]
    dis_ref, w_ref, b_ref = refs[2 + C:5 + C]
    outs = refs[5 + C:]
    d = dis_ref[...][:, 0:1]
    du2 = jnp.concatenate(
        [d * (pr[...][0] + pr[...][1]) for pr in p_refs], axis=1)
    w = w_ref[...]
    w0 = w[0] - w[2]
    w1 = w[1]
    w2 = 2.0 * w[2]
    acc = jnp.dot(h_ref[...], w0, preferred_element_type=jnp.float32)
    acc = acc - jnp.dot(du_ref[...], w1, preferred_element_type=jnp.float32)
    acc = acc + jnp.dot(du2, w2, preferred_element_type=jnp.float32)
    acc = jnp.maximum(acc + b_ref[...], 0.0)
    outs[0][...] = acc
    if emit_hs:
      hs = d * acc
      outs[1][...] = jnp.stack(
          [hs[:, c * FC:(c + 1) * FC] for c in range(C2)], axis=0)

  out_specs = [pl.BlockSpec((RB, Fout), lambda i: (i, 0))]
  out_shape = [jax.ShapeDtypeStruct((N, Fout), jnp.float32)]
  if emit_hs:
    out_specs.append(pl.BlockSpec((C2, RB, FC), lambda i: (0, i, 0)))
    out_shape.append(jax.ShapeDtypeStruct((C2, N, FC), jnp.float32))

  return pl.pallas_call(
      body,
      grid=(N // RB,),
      in_specs=[
          pl.BlockSpec((RB, Fin), lambda i: (i, 0)),
          pl.BlockSpec((RB, Fin), lambda i: (i, 0)),
      ]
      + [pl.BlockSpec((2, RB, FC), lambda i: (0, i, 0))] * C
      + [
          pl.BlockSpec((RB, 16), lambda i: (i, 0)),
          pl.BlockSpec((3, Fin, Fout), lambda i: (0, 0, 0)),
          pl.BlockSpec((1, Fout), lambda i: (0, 0)),
      ],
      out_specs=out_specs,
      out_shape=out_shape,
  )(h, du1, *partials2, dis16, Wl, bl.reshape(1, Fout))


def kernel(x, edge_index, W1, b1, W2, b2, W3, b3):
  src = edge_index[0]
  dst = edge_index[1]
  pad = EP - E
  padi = jnp.arange(pad, dtype=jnp.int32)
  gidx = jnp.concatenate([src, padi % N])
  sidx = jnp.concatenate([dst, N + padi % (NP - N)])
  didx = jnp.concatenate([src, N + padi % (NP - N)])

  zfc = jnp.zeros((ZR, FC), jnp.float32)

  dp = _sc_segsum(jnp.ones((N, FC), jnp.float32), gidx, didx, zfc)
  dis16, hs0 = _tc_prep(x, dp)

  h = x
  hs_chunks = hs0[None]
  for Wl, bl in ((W1, b1), (W2, b2), (W3, b3)):
    C = hs_chunks.shape[0]
    p1 = [_sc_segsum(hs_chunks[c], gidx, sidx, zfc) for c in range(C)]
    du1, v2c = _tc_mid(p1, dis16)
    p2 = [_sc_segsum(v2c[c], gidx, sidx, zfc) for c in range(C)]
    emit_hs = Wl is not W3
    res = _tc_layer_out(h, du1, p2, dis16, Wl, bl, emit_hs)
    if emit_hs:
      h, hs_chunks = res
    else:
      h = res[0]
  return h
